# baseline (speedup 1.0000x reference)
import jax
import jax.numpy as jnp
from jax.experimental import pallas as pl

TAU = 0.07


def _stage1_body(mask_ref, fq_ref, fk_ref, s_ref, cnt_ref):
    bi = pl.program_id(0)
    ci = pl.program_id(1)
    M, HC, w = mask_ref.shape[1:]
    c2 = fq_ref.shape[1] * 2
    m = mask_ref[0].astype(jnp.float32).reshape(M, HC * w)
    cat = jnp.concatenate([fq_ref[0], fk_ref[0]], axis=0).reshape(c2, HC * w)
    dn = (((1,), (1,)), ((), ()))
    s = jax.lax.dot_general(m, cat, dn, preferred_element_type=jnp.float32)
    c = jnp.sum(m, axis=1)

    @pl.when(ci == 0)
    def _():
        s_ref[bi] = s
        cnt_ref[bi] = c

    @pl.when(ci != 0)
    def _():
        s_ref[bi] += s
        cnt_ref[bi] += c


def _stage2_body(s_ref, cnt_ref, out_ref):
    s = s_ref[...]
    b, M, c2 = s.shape
    c = c2 // 2
    N = M * b
    cnt = jnp.maximum(cnt_ref[...], 1.0)[..., None]
    mean = s / cnt
    mean = jnp.transpose(mean, (1, 0, 2)).reshape(N, c2)
    mq = mean[:, :c]
    mk = mean[:, c:]
    pad = mk[:, 0:1] != 0.0
    nq = mq / jnp.maximum(
        jnp.sqrt(jnp.sum(mq * mq, axis=1, keepdims=True)), 1e-12)
    nk = mk / jnp.maximum(
        jnp.sqrt(jnp.sum(mk * mk, axis=1, keepdims=True)), 1e-12)
    dn = (((1,), (1,)), ((), ()))
    sim = jax.lax.dot_general(nk, nq, dn,
                              preferred_element_type=jnp.float32) / TAU
    mx = jnp.max(sim, axis=1, keepdims=True)
    lse = mx + jnp.log(jnp.sum(jnp.exp(sim - mx), axis=1, keepdims=True))
    ri = jax.lax.broadcasted_iota(jnp.int32, (N, N), 0)
    ci = jax.lax.broadcasted_iota(jnp.int32, (N, N), 1)
    diag = jnp.sum(jnp.where(ri == ci, sim, 0.0), axis=1, keepdims=True)
    ce = lse - diag
    valid = jnp.where(pad, 1.0, 0.0)
    num = jnp.sum(ce * valid, axis=(0, 1), keepdims=True)
    den = jnp.maximum(jnp.sum(valid, axis=(0, 1), keepdims=True), 1.0)
    out_ref[...] = num / den


def kernel(features_q, features_k, pos_region_ranges):
    b, c, h, w = features_q.shape
    M = pos_region_ranges.shape[1]

    HC = 16
    s, cnt = pl.pallas_call(
        _stage1_body,
        grid=(b, h // HC),
        in_specs=[
            pl.BlockSpec((1, M, HC, w), lambda bi, ci: (bi, 0, ci, 0)),
            pl.BlockSpec((1, c, HC, w), lambda bi, ci: (bi, 0, ci, 0)),
            pl.BlockSpec((1, c, HC, w), lambda bi, ci: (bi, 0, ci, 0)),
        ],
        out_specs=[
            pl.BlockSpec((b, M, 2 * c), lambda bi, ci: (0, 0, 0)),
            pl.BlockSpec((b, M), lambda bi, ci: (0, 0)),
        ],
        out_shape=[
            jax.ShapeDtypeStruct((b, M, 2 * c), jnp.float32),
            jax.ShapeDtypeStruct((b, M), jnp.float32),
        ],
    )(pos_region_ranges, features_q, features_k)

    loss = pl.pallas_call(
        _stage2_body,
        out_shape=jax.ShapeDtypeStruct((1, 1), jnp.float32),
    )(s, cnt)
    return loss[0, 0]


# speedup vs baseline: 1.2405x; 1.2405x over previous
import jax
import jax.numpy as jnp
from jax.experimental import pallas as pl

TAU = 0.07


def _stage1_body(mask_ref, fq_ref, fk_ref, s_ref, cnt_ref):
    bi = pl.program_id(0)
    ci = pl.program_id(1)
    M, HC, w = mask_ref.shape[1:]
    c2 = fq_ref.shape[1] * 2
    m = mask_ref[0].astype(jnp.float32).reshape(M, HC * w)
    cat = jnp.concatenate([fq_ref[0], fk_ref[0]], axis=0).reshape(c2, HC * w)
    dn = (((1,), (1,)), ((), ()))
    s = jax.lax.dot_general(m, cat, dn, preferred_element_type=jnp.float32)
    c = jnp.sum(m, axis=1)

    @pl.when(ci == 0)
    def _():
        s_ref[bi] = s
        cnt_ref[bi] = c

    @pl.when(ci != 0)
    def _():
        s_ref[bi] += s
        cnt_ref[bi] += c


def _stage2_body(s_ref, cnt_ref, out_ref):
    s = s_ref[...]
    b, M, c2 = s.shape
    c = c2 // 2
    N = M * b
    cnt = jnp.maximum(cnt_ref[...], 1.0)[..., None]
    mean = s / cnt
    mean = jnp.transpose(mean, (1, 0, 2)).reshape(N, c2)
    mq = mean[:, :c]
    mk = mean[:, c:]
    pad = mk[:, 0:1] != 0.0
    nq = mq / jnp.maximum(
        jnp.sqrt(jnp.sum(mq * mq, axis=1, keepdims=True)), 1e-12)
    nk = mk / jnp.maximum(
        jnp.sqrt(jnp.sum(mk * mk, axis=1, keepdims=True)), 1e-12)
    dn = (((1,), (1,)), ((), ()))
    sim = jax.lax.dot_general(nk, nq, dn,
                              preferred_element_type=jnp.float32) / TAU
    mx = jnp.max(sim, axis=1, keepdims=True)
    lse = mx + jnp.log(jnp.sum(jnp.exp(sim - mx), axis=1, keepdims=True))
    ri = jax.lax.broadcasted_iota(jnp.int32, (N, N), 0)
    ci = jax.lax.broadcasted_iota(jnp.int32, (N, N), 1)
    diag = jnp.sum(jnp.where(ri == ci, sim, 0.0), axis=1, keepdims=True)
    ce = lse - diag
    valid = jnp.where(pad, 1.0, 0.0)
    num = jnp.sum(ce * valid, axis=(0, 1), keepdims=True)
    den = jnp.maximum(jnp.sum(valid, axis=(0, 1), keepdims=True), 1.0)
    out_ref[...] = num / den


def kernel(features_q, features_k, pos_region_ranges):
    b, c, h, w = features_q.shape
    M = pos_region_ranges.shape[1]

    HC = 32
    s, cnt = pl.pallas_call(
        _stage1_body,
        grid=(b, h // HC),
        in_specs=[
            pl.BlockSpec((1, M, HC, w), lambda bi, ci: (bi, 0, ci, 0)),
            pl.BlockSpec((1, c, HC, w), lambda bi, ci: (bi, 0, ci, 0)),
            pl.BlockSpec((1, c, HC, w), lambda bi, ci: (bi, 0, ci, 0)),
        ],
        out_specs=[
            pl.BlockSpec((b, M, 2 * c), lambda bi, ci: (0, 0, 0)),
            pl.BlockSpec((b, M), lambda bi, ci: (0, 0)),
        ],
        out_shape=[
            jax.ShapeDtypeStruct((b, M, 2 * c), jnp.float32),
            jax.ShapeDtypeStruct((b, M), jnp.float32),
        ],
    )(pos_region_ranges, features_q, features_k)

    loss = pl.pallas_call(
        _stage2_body,
        out_shape=jax.ShapeDtypeStruct((1, 1), jnp.float32),
    )(s, cnt)
    return loss[0, 0]
